# baseline (speedup 1.0000x reference)
import functools

import jax
import jax.numpy as jnp
from jax import lax
from jax.experimental import pallas as pl
from jax.experimental.pallas import tpu as pltpu
from jax.experimental.pallas import tpu_sc as plsc

N = 10000
F = 128
HID = 256
NC = 2
NS = 16
NW = NC * NS
CHB = 128
NCH = 81
NCHS = 88
PT = NCH * CHB
E_PAD = NW * PT
HALVES = ((0, 48, 48), (48, 36, 40))
NACC = 10240
DPT = NACC // NS
NACC_D = 10240
DPT_D = NACC_D // NS
DUMMY = N

_mesh = plsc.VectorSubcoreMesh(core_axis_name="c", subcore_axis_name="s")


@functools.partial(
    pl.kernel,
    out_type=jax.ShapeDtypeStruct((NC * NACC_D,), jnp.float32),
    mesh=_mesh,
    scratch_types=[
        pltpu.VMEM((NCHS, CHB), jnp.int32),
        pltpu.VMEM((CHB,), jnp.float32),
        pltpu.VMEM_SHARED((NACC_D,), jnp.float32),
    ],
)
def _deg_call(dst_hbm, zd_hbm, out_hbm, dst_all, ones_v, acc):
    c = lax.axis_index("c")
    s = lax.axis_index("s")
    wid = s * NC + c
    pltpu.sync_copy(zd_hbm, acc.at[pl.ds(s * DPT_D, DPT_D)])
    pltpu.sync_copy(dst_hbm.at[pl.ds(wid * NCHS, NCHS)], dst_all)
    ones16 = jnp.ones((16,), jnp.float32)
    for i in range(CHB // 16):
        ones_v[pl.ds(i * 16, 16)] = ones16
    plsc.subcore_barrier()

    @pl.loop(0, NCH)
    def _(r):
        pltpu.sync_copy(ones_v, acc.at[dst_all.at[r]], add=True)

    plsc.subcore_barrier()
    pltpu.sync_copy(acc.at[pl.ds(s * DPT_D, DPT_D)],
                    out_hbm.at[pl.ds(c * NACC_D + s * DPT_D, DPT_D)])


@functools.partial(
    pl.kernel,
    out_type=jax.ShapeDtypeStruct((NC * NACC, F), jnp.float32),
    mesh=_mesh,
    scratch_types=[
        pltpu.VMEM((NCHS, CHB), jnp.int32),
        pltpu.VMEM((NCHS, CHB), jnp.int32),
        pltpu.VMEM((CHB, F), jnp.float32),
        pltpu.VMEM_SHARED((NACC, F), jnp.float32),
        pltpu.SemaphoreType.DMA,
    ],
)
def _agg_call(tab_hbm, src_hbm, dst_hbm, zc_hbm, out_hbm,
              src_all, dst_all, rows, acc, gsem):
    c = lax.axis_index("c")
    s = lax.axis_index("s")
    wid = s * NC + c
    pltpu.sync_copy(zc_hbm, acc.at[pl.ds(s * DPT, DPT)])
    plsc.subcore_barrier()

    for base, nk, nrows in HALVES:
        pltpu.sync_copy(src_hbm.at[pl.ds(wid * NCHS + base, nrows)],
                        src_all.at[pl.ds(0, nrows)])
        pltpu.sync_copy(dst_hbm.at[pl.ds(wid * NCHS + base, nrows)],
                        dst_all.at[pl.ds(0, nrows)])

        @pl.loop(0, nk)
        def _(j):
            pltpu.async_copy(tab_hbm.at[src_all.at[j]], rows, gsem).wait()
            pltpu.sync_copy(rows, acc.at[dst_all.at[j]], add=True)

    plsc.subcore_barrier()
    pltpu.sync_copy(acc.at[pl.ds(s * DPT, DPT)],
                    out_hbm.at[pl.ds(c * NACC + s * DPT, DPT)])


def _dinv_body(degp_ref, dinv_ref):
    deg = degp_ref[0] + degp_ref[1]
    dinv_ref[...] = jnp.where(
        deg > 0, lax.rsqrt(jnp.maximum(deg, 1e-12)), 0.0)


_dinv_call = pl.pallas_call(
    _dinv_body,
    out_shape=jax.ShapeDtypeStruct((NACC_D // 128, 128), jnp.float32),
)

_BR = 2000
_GRID = N // _BR


def _scale_body(x_ref, d_ref, o_ref):
    o_ref[...] = x_ref[...] * d_ref[...]


_scale_call = pl.pallas_call(
    _scale_body,
    grid=(_GRID,),
    in_specs=[
        pl.BlockSpec((_BR, F), lambda i: (i, 0)),
        pl.BlockSpec((_BR, 1), lambda i: (i, 0)),
    ],
    out_specs=pl.BlockSpec((_BR, F), lambda i: (i, 0)),
    out_shape=jax.ShapeDtypeStruct((N, F), jnp.float32),
)


def _mlp_body(p_ref, d_ref, w1_ref, b1_ref, w2_ref, y_ref):
    agg = (p_ref[0] + p_ref[1]) * d_ref[...]
    h = jnp.dot(agg, w1_ref[...], preferred_element_type=jnp.float32)
    h = jnp.maximum(h + b1_ref[...], 0.0)
    y = jnp.dot(h, w2_ref[...], preferred_element_type=jnp.float32)
    y_ref[...] = y * d_ref[...]


_mlp_call = pl.pallas_call(
    _mlp_body,
    grid=(_GRID,),
    in_specs=[
        pl.BlockSpec((2, _BR, F), lambda i: (0, i, 0)),
        pl.BlockSpec((_BR, 1), lambda i: (i, 0)),
        pl.BlockSpec((F, HID), lambda i: (0, 0)),
        pl.BlockSpec((1, HID), lambda i: (0, 0)),
        pl.BlockSpec((HID, F), lambda i: (0, 0)),
    ],
    out_specs=pl.BlockSpec((_BR, F), lambda i: (i, 0)),
    out_shape=jax.ShapeDtypeStruct((N, F), jnp.float32),
)


def _final_body(p_ref, d_ref, b2_ref, o_ref):
    o_ref[...] = (p_ref[0] + p_ref[1]) * d_ref[...] + b2_ref[...]


_final_call = pl.pallas_call(
    _final_body,
    grid=(_GRID,),
    in_specs=[
        pl.BlockSpec((2, _BR, F), lambda i: (0, i, 0)),
        pl.BlockSpec((_BR, 1), lambda i: (i, 0)),
        pl.BlockSpec((1, F), lambda i: (0, 0)),
    ],
    out_specs=pl.BlockSpec((_BR, F), lambda i: (i, 0)),
    out_shape=jax.ShapeDtypeStruct((N, F), jnp.float32),
)


def kernel(x, edge_index, W1, b1, W2, b2):
    loop = jnp.arange(N, dtype=jnp.int32)
    npad = E_PAD - (edge_index.shape[1] + N)
    src = jnp.concatenate(
        [edge_index[0], loop, jnp.zeros((npad,), jnp.int32)])
    dst = jnp.concatenate(
        [edge_index[1], loop, jnp.full((npad,), DUMMY, jnp.int32)])
    src2d = jnp.concatenate(
        [src.reshape(NW, NCH, CHB),
         jnp.zeros((NW, NCHS - NCH, CHB), jnp.int32)],
        axis=1).reshape(NW * NCHS, CHB)
    dst2d = jnp.concatenate(
        [dst.reshape(NW, NCH, CHB),
         jnp.full((NW, NCHS - NCH, CHB), DUMMY, jnp.int32)],
        axis=1).reshape(NW * NCHS, CHB)
    zd = jnp.zeros((DPT_D,), jnp.float32)
    zc = jnp.zeros((DPT, F), jnp.float32)

    degp = _deg_call(dst2d, zd)
    dinv2d = _dinv_call(degp.reshape(2, NACC_D // 128, 128))
    dinv_col = dinv2d.reshape(NACC_D, 1)[:N]
    xs = _scale_call(x, dinv_col)
    p1 = _agg_call(xs, src2d, dst2d, zc).reshape(2, NACC, F)
    y = _mlp_call(p1, dinv_col, W1, b1.reshape(1, HID), W2)
    p2 = _agg_call(y, src2d, dst2d, zc).reshape(2, NACC, F)
    return _final_call(p2, dinv_col, b2.reshape(1, F))


# speedup vs baseline: 3.3991x; 3.3991x over previous
import functools

import jax
import jax.numpy as jnp
from jax import lax
from jax.experimental import pallas as pl
from jax.experimental.pallas import tpu as pltpu
from jax.experimental.pallas import tpu_sc as plsc

N = 10000
F = 128
HID = 256
NC = 2
NS = 16
NW = NC * NS
CHB = 128
NCH = 81
NCHS = 88
PT = NCH * CHB
E_PAD = NW * PT
HALVES = ((0, 48, 48), (48, 36, 40))
NACC = 10240
DPT = NACC // NS
NACC_D = 10240
DPT_D = NACC_D // NS
DUMMY = N

_mesh = plsc.VectorSubcoreMesh(core_axis_name="c", subcore_axis_name="s")


@functools.partial(
    pl.kernel,
    out_type=jax.ShapeDtypeStruct((NC * NACC_D,), jnp.float32),
    mesh=_mesh,
    scratch_types=[
        pltpu.VMEM((NCHS, CHB), jnp.int32),
        pltpu.VMEM((CHB,), jnp.float32),
        pltpu.VMEM_SHARED((NACC_D,), jnp.float32),
    ],
)
def _deg_call(dst_hbm, zd_hbm, out_hbm, dst_all, ones_v, acc):
    c = lax.axis_index("c")
    s = lax.axis_index("s")
    wid = s * NC + c
    pltpu.sync_copy(zd_hbm, acc.at[pl.ds(s * DPT_D, DPT_D)])
    pltpu.sync_copy(dst_hbm.at[pl.ds(wid * NCHS, NCHS)], dst_all)
    ones16 = jnp.ones((16,), jnp.float32)
    for i in range(CHB // 16):
        ones_v[pl.ds(i * 16, 16)] = ones16
    plsc.subcore_barrier()

    @pl.loop(0, NCH)
    def _(r):
        pltpu.sync_copy(ones_v, acc.at[dst_all.at[r]], add=True)

    plsc.subcore_barrier()
    pltpu.sync_copy(acc.at[pl.ds(s * DPT_D, DPT_D)],
                    out_hbm.at[pl.ds(c * NACC_D + s * DPT_D, DPT_D)])


@functools.partial(
    pl.kernel,
    out_type=jax.ShapeDtypeStruct((NC * NACC, F), jnp.float32),
    mesh=_mesh,
    scratch_types=[
        pltpu.VMEM((NCHS, CHB), jnp.int32),
        pltpu.VMEM((NCHS, CHB), jnp.int32),
        pltpu.VMEM((CHB, F), jnp.float32),
        pltpu.VMEM_SHARED((NACC, F), jnp.float32),
        pltpu.SemaphoreType.DMA,
    ],
)
def _agg_call(tab_hbm, src_hbm, dst_hbm, zc_hbm, out_hbm,
              src_all, dst_all, rows, acc, gsem):
    c = lax.axis_index("c")
    s = lax.axis_index("s")
    wid = s * NC + c
    pltpu.sync_copy(zc_hbm, acc.at[pl.ds(s * DPT, DPT)])
    pltpu.sync_copy(src_hbm.at[pl.ds(wid * NCHS, NCHS)], src_all)
    pltpu.sync_copy(dst_hbm.at[pl.ds(wid * NCHS, NCHS)], dst_all)
    plsc.subcore_barrier()

    @pl.loop(0, NCH)
    def _(j):
        pltpu.async_copy(tab_hbm.at[src_all.at[j]], rows, gsem).wait()
        pltpu.sync_copy(rows, acc.at[dst_all.at[j]], add=True)

    plsc.subcore_barrier()
    pltpu.sync_copy(acc.at[pl.ds(s * DPT, DPT)],
                    out_hbm.at[pl.ds(c * NACC + s * DPT, DPT)])


def _dinv_body(degp_ref, dinv_ref):
    deg = degp_ref[0] + degp_ref[1]
    dinv_ref[...] = jnp.where(
        deg > 0, lax.rsqrt(jnp.maximum(deg, 1e-12)), 0.0)


_dinv_call = pl.pallas_call(
    _dinv_body,
    out_shape=jax.ShapeDtypeStruct((NACC_D // 128, 128), jnp.float32),
)

_BR = 2000
_GRID = N // _BR


def _scale_body(x_ref, d_ref, o_ref):
    o_ref[...] = x_ref[...] * d_ref[...]


_scale_call = pl.pallas_call(
    _scale_body,
    grid=(_GRID,),
    in_specs=[
        pl.BlockSpec((_BR, F), lambda i: (i, 0)),
        pl.BlockSpec((_BR, 1), lambda i: (i, 0)),
    ],
    out_specs=pl.BlockSpec((_BR, F), lambda i: (i, 0)),
    out_shape=jax.ShapeDtypeStruct((N, F), jnp.float32),
)


def _mlp_body(p_ref, d_ref, w1_ref, b1_ref, w2_ref, y_ref):
    agg = (p_ref[0] + p_ref[1]) * d_ref[...]
    h = jnp.dot(agg, w1_ref[...], preferred_element_type=jnp.float32)
    h = jnp.maximum(h + b1_ref[...], 0.0)
    y = jnp.dot(h, w2_ref[...], preferred_element_type=jnp.float32)
    y_ref[...] = y * d_ref[...]


_mlp_call = pl.pallas_call(
    _mlp_body,
    grid=(_GRID,),
    in_specs=[
        pl.BlockSpec((2, _BR, F), lambda i: (0, i, 0)),
        pl.BlockSpec((_BR, 1), lambda i: (i, 0)),
        pl.BlockSpec((F, HID), lambda i: (0, 0)),
        pl.BlockSpec((1, HID), lambda i: (0, 0)),
        pl.BlockSpec((HID, F), lambda i: (0, 0)),
    ],
    out_specs=pl.BlockSpec((_BR, F), lambda i: (i, 0)),
    out_shape=jax.ShapeDtypeStruct((N, F), jnp.float32),
)


def _final_body(p_ref, d_ref, b2_ref, o_ref):
    o_ref[...] = (p_ref[0] + p_ref[1]) * d_ref[...] + b2_ref[...]


_final_call = pl.pallas_call(
    _final_body,
    grid=(_GRID,),
    in_specs=[
        pl.BlockSpec((2, _BR, F), lambda i: (0, i, 0)),
        pl.BlockSpec((_BR, 1), lambda i: (i, 0)),
        pl.BlockSpec((1, F), lambda i: (0, 0)),
    ],
    out_specs=pl.BlockSpec((_BR, F), lambda i: (i, 0)),
    out_shape=jax.ShapeDtypeStruct((N, F), jnp.float32),
)


def kernel(x, edge_index, W1, b1, W2, b2):
    loop = jnp.arange(N, dtype=jnp.int32)
    npad = E_PAD - (edge_index.shape[1] + N)
    pad_ids = jnp.arange(npad, dtype=jnp.int32)
    src = jnp.concatenate([edge_index[0], loop, pad_ids % N])
    dst = jnp.concatenate([edge_index[1], loop, DUMMY + pad_ids % (NACC - N)])
    src2d = jnp.concatenate(
        [src.reshape(NW, NCH, CHB),
         jnp.zeros((NW, NCHS - NCH, CHB), jnp.int32)],
        axis=1).reshape(NW * NCHS, CHB)
    dst2d = jnp.concatenate(
        [dst.reshape(NW, NCH, CHB),
         jnp.full((NW, NCHS - NCH, CHB), DUMMY, jnp.int32)],
        axis=1).reshape(NW * NCHS, CHB)
    zd = jnp.zeros((DPT_D,), jnp.float32)
    zc = jnp.zeros((DPT, F), jnp.float32)

    degp = _deg_call(dst2d, zd)
    dinv2d = _dinv_call(degp.reshape(2, NACC_D // 128, 128))
    dinv_col = dinv2d.reshape(NACC_D, 1)[:N]
    xs = _scale_call(x, dinv_col)
    p1 = _agg_call(xs, src2d, dst2d, zc).reshape(2, NACC, F)
    y = _mlp_call(p1, dinv_col, W1, b1.reshape(1, HID), W2)
    p2 = _agg_call(y, src2d, dst2d, zc).reshape(2, NACC, F)
    return _final_call(p2, dinv_col, b2.reshape(1, F))


# speedup vs baseline: 3.7935x; 1.1160x over previous
import functools

import jax
import jax.numpy as jnp
from jax import lax
from jax.experimental import pallas as pl
from jax.experimental.pallas import tpu as pltpu
from jax.experimental.pallas import tpu_sc as plsc

N = 10000
F = 128
HID = 256
NC = 2
NS = 16
NW = NC * NS
CHB = 128
NCH = 81
NCHS = 88
PT = NCH * CHB
E_PAD = NW * PT
HALVES = ((0, 48, 48), (48, 36, 40))
NACC = 10240
DPT = NACC // NS
NACC_D = 10240
DPT_D = NACC_D // NS
DUMMY = N

_mesh = plsc.VectorSubcoreMesh(core_axis_name="c", subcore_axis_name="s")


@functools.partial(
    pl.kernel,
    out_type=jax.ShapeDtypeStruct((NC * NACC_D,), jnp.float32),
    mesh=_mesh,
    scratch_types=[
        pltpu.VMEM((NCHS, CHB), jnp.int32),
        pltpu.VMEM((CHB,), jnp.float32),
        pltpu.VMEM_SHARED((NACC_D,), jnp.float32),
    ],
)
def _deg_call(dst_hbm, zd_hbm, out_hbm, dst_all, ones_v, acc):
    c = lax.axis_index("c")
    s = lax.axis_index("s")
    wid = s * NC + c
    pltpu.sync_copy(zd_hbm, acc.at[pl.ds(s * DPT_D, DPT_D)])
    pltpu.sync_copy(dst_hbm.at[pl.ds(wid * NCHS, NCHS)], dst_all)
    ones16 = jnp.ones((16,), jnp.float32)
    for i in range(CHB // 16):
        ones_v[pl.ds(i * 16, 16)] = ones16
    plsc.subcore_barrier()

    @pl.loop(0, NCH)
    def _(r):
        pltpu.sync_copy(ones_v, acc.at[dst_all.at[r]], add=True)

    plsc.subcore_barrier()
    pltpu.sync_copy(acc.at[pl.ds(s * DPT_D, DPT_D)],
                    out_hbm.at[pl.ds(c * NACC_D + s * DPT_D, DPT_D)])


@functools.partial(
    pl.kernel,
    out_type=jax.ShapeDtypeStruct((NC * NACC, F), jnp.float32),
    mesh=_mesh,
    scratch_types=[
        pltpu.VMEM((48, CHB), jnp.int32),
        pltpu.VMEM((48, CHB), jnp.int32),
        pltpu.VMEM((CHB, F), jnp.float32),
        pltpu.VMEM((CHB, F), jnp.float32),
        pltpu.VMEM_SHARED((NACC, F), jnp.float32),
        pltpu.SemaphoreType.DMA,
        pltpu.SemaphoreType.DMA,
        pltpu.SemaphoreType.DMA,
        pltpu.SemaphoreType.DMA,
    ],
)
def _agg_call(tab_hbm, src_hbm, dst_hbm, zc_hbm, out_hbm,
              sidx, didx, rows0, rows1, acc, gsem0, gsem1, ssem0, ssem1):
    c = lax.axis_index("c")
    s = lax.axis_index("s")
    wid = s * NC + c
    pltpu.sync_copy(zc_hbm, acc.at[pl.ds(s * DPT, DPT)])
    plsc.subcore_barrier()

    for base, nk, nrows in HALVES:
        pltpu.sync_copy(src_hbm.at[pl.ds(wid * NCHS + base, nrows)],
                        sidx.at[pl.ds(0, nrows)])
        pltpu.sync_copy(dst_hbm.at[pl.ds(wid * NCHS + base, nrows)],
                        didx.at[pl.ds(0, nrows)])

        @pl.loop(0, nk // 2)
        def _(r):
            k = r * 2
            g0 = pltpu.async_copy(tab_hbm.at[sidx.at[k]], rows0, gsem0)
            g1 = pltpu.async_copy(tab_hbm.at[sidx.at[k + 1]], rows1, gsem1)
            g0.wait()
            s0 = pltpu.async_copy(rows0, acc.at[didx.at[k]], ssem0,
                                  add=True)
            g1.wait()
            s1 = pltpu.async_copy(rows1, acc.at[didx.at[k + 1]], ssem1,
                                  add=True)
            s0.wait()
            s1.wait()

    plsc.subcore_barrier()
    pltpu.sync_copy(acc.at[pl.ds(s * DPT, DPT)],
                    out_hbm.at[pl.ds(c * NACC + s * DPT, DPT)])


def _dinv_body(degp_ref, dinv_ref):
    deg = degp_ref[0] + degp_ref[1]
    dinv_ref[...] = jnp.where(
        deg > 0, lax.rsqrt(jnp.maximum(deg, 1e-12)), 0.0)


_dinv_call = pl.pallas_call(
    _dinv_body,
    out_shape=jax.ShapeDtypeStruct((NACC_D // 128, 128), jnp.float32),
)

_BR = 2000
_GRID = N // _BR


def _scale_body(x_ref, d_ref, o_ref):
    o_ref[...] = x_ref[...] * d_ref[...]


_scale_call = pl.pallas_call(
    _scale_body,
    grid=(_GRID,),
    in_specs=[
        pl.BlockSpec((_BR, F), lambda i: (i, 0)),
        pl.BlockSpec((_BR, 1), lambda i: (i, 0)),
    ],
    out_specs=pl.BlockSpec((_BR, F), lambda i: (i, 0)),
    out_shape=jax.ShapeDtypeStruct((N, F), jnp.float32),
)


def _mlp_body(p_ref, d_ref, w1_ref, b1_ref, w2_ref, y_ref):
    agg = (p_ref[0] + p_ref[1]) * d_ref[...]
    h = jnp.dot(agg, w1_ref[...], preferred_element_type=jnp.float32)
    h = jnp.maximum(h + b1_ref[...], 0.0)
    y = jnp.dot(h, w2_ref[...], preferred_element_type=jnp.float32)
    y_ref[...] = y * d_ref[...]


_mlp_call = pl.pallas_call(
    _mlp_body,
    grid=(_GRID,),
    in_specs=[
        pl.BlockSpec((2, _BR, F), lambda i: (0, i, 0)),
        pl.BlockSpec((_BR, 1), lambda i: (i, 0)),
        pl.BlockSpec((F, HID), lambda i: (0, 0)),
        pl.BlockSpec((1, HID), lambda i: (0, 0)),
        pl.BlockSpec((HID, F), lambda i: (0, 0)),
    ],
    out_specs=pl.BlockSpec((_BR, F), lambda i: (i, 0)),
    out_shape=jax.ShapeDtypeStruct((N, F), jnp.float32),
)


def _final_body(p_ref, d_ref, b2_ref, o_ref):
    o_ref[...] = (p_ref[0] + p_ref[1]) * d_ref[...] + b2_ref[...]


_final_call = pl.pallas_call(
    _final_body,
    grid=(_GRID,),
    in_specs=[
        pl.BlockSpec((2, _BR, F), lambda i: (0, i, 0)),
        pl.BlockSpec((_BR, 1), lambda i: (i, 0)),
        pl.BlockSpec((1, F), lambda i: (0, 0)),
    ],
    out_specs=pl.BlockSpec((_BR, F), lambda i: (i, 0)),
    out_shape=jax.ShapeDtypeStruct((N, F), jnp.float32),
)


def kernel(x, edge_index, W1, b1, W2, b2):
    loop = jnp.arange(N, dtype=jnp.int32)
    npad = E_PAD - (edge_index.shape[1] + N)
    pad_ids = jnp.arange(npad, dtype=jnp.int32)
    src = jnp.concatenate([edge_index[0], loop, pad_ids % N])
    dst = jnp.concatenate([edge_index[1], loop, DUMMY + pad_ids % (NACC - N)])
    nslabpad = NW * (NCHS - NCH) * CHB
    slab_ids = jnp.arange(nslabpad, dtype=jnp.int32)
    src2d = jnp.concatenate(
        [src.reshape(NW, NCH, CHB),
         (slab_ids % N).reshape(NW, NCHS - NCH, CHB)],
        axis=1).reshape(NW * NCHS, CHB)
    dst2d = jnp.concatenate(
        [dst.reshape(NW, NCH, CHB),
         (DUMMY + slab_ids % (NACC - N)).reshape(NW, NCHS - NCH, CHB)],
        axis=1).reshape(NW * NCHS, CHB)
    zd = jnp.zeros((DPT_D,), jnp.float32)
    zc = jnp.zeros((DPT, F), jnp.float32)

    degp = _deg_call(dst2d, zd)
    dinv2d = _dinv_call(degp.reshape(2, NACC_D // 128, 128))
    dinv_col = dinv2d.reshape(NACC_D, 1)[:N]
    xs = _scale_call(x, dinv_col)
    p1 = _agg_call(xs, src2d, dst2d, zc).reshape(2, NACC, F)
    y = _mlp_call(p1, dinv_col, W1, b1.reshape(1, HID), W2)
    p2 = _agg_call(y, src2d, dst2d, zc).reshape(2, NACC, F)
    return _final_call(p2, dinv_col, b2.reshape(1, F))


# speedup vs baseline: 3.8428x; 1.0130x over previous
import functools

import jax
import jax.numpy as jnp
from jax import lax
from jax.experimental import pallas as pl
from jax.experimental.pallas import tpu as pltpu
from jax.experimental.pallas import tpu_sc as plsc

N = 10000
F = 128
HID = 256
NC = 2
NS = 16
NW = NC * NS
CHB = 128
NCH = 81
NCHS = 88
PT = NCH * CHB
E_PAD = NW * PT
HALVES = ((0, 48, 48), (48, 34, 40))
NACC = 10240
DPT = NACC // NS
NACC_D = 10240
DPT_D = NACC_D // NS
DUMMY = N

_mesh = plsc.VectorSubcoreMesh(core_axis_name="c", subcore_axis_name="s")


@functools.partial(
    pl.kernel,
    out_type=jax.ShapeDtypeStruct((NC * NACC_D,), jnp.float32),
    mesh=_mesh,
    scratch_types=[
        pltpu.VMEM((NCHS, CHB), jnp.int32),
        pltpu.VMEM((CHB,), jnp.float32),
        pltpu.VMEM_SHARED((NACC_D,), jnp.float32),
    ],
)
def _deg_call(dst_hbm, zd_hbm, out_hbm, dst_all, ones_v, acc):
    c = lax.axis_index("c")
    s = lax.axis_index("s")
    wid = s * NC + c
    pltpu.sync_copy(zd_hbm, acc.at[pl.ds(s * DPT_D, DPT_D)])
    pltpu.sync_copy(dst_hbm.at[pl.ds(wid * NCHS, NCHS)], dst_all)
    ones16 = jnp.ones((16,), jnp.float32)
    for i in range(CHB // 16):
        ones_v[pl.ds(i * 16, 16)] = ones16
    plsc.subcore_barrier()

    @pl.loop(0, NCH)
    def _(r):
        pltpu.sync_copy(ones_v, acc.at[dst_all.at[r]], add=True)

    plsc.subcore_barrier()
    pltpu.sync_copy(acc.at[pl.ds(s * DPT_D, DPT_D)],
                    out_hbm.at[pl.ds(c * NACC_D + s * DPT_D, DPT_D)])


@functools.partial(
    pl.kernel,
    out_type=jax.ShapeDtypeStruct((NC * NACC, F), jnp.float32),
    mesh=_mesh,
    scratch_types=[
        pltpu.VMEM((48, CHB), jnp.int32),
        pltpu.VMEM((48, CHB), jnp.int32),
        pltpu.VMEM((CHB, F), jnp.float32),
        pltpu.VMEM((CHB, F), jnp.float32),
        pltpu.VMEM_SHARED((NACC, F), jnp.float32),
        pltpu.SemaphoreType.DMA,
        pltpu.SemaphoreType.DMA,
        pltpu.SemaphoreType.DMA,
        pltpu.SemaphoreType.DMA,
    ],
)
def _agg_call(tab_hbm, src_hbm, dst_hbm, zc_hbm, out_hbm,
              sidx, didx, rows0, rows1, acc, gsem0, gsem1, ssem0, ssem1):
    c = lax.axis_index("c")
    s = lax.axis_index("s")
    wid = s * NC + c
    pltpu.sync_copy(zc_hbm, acc.at[pl.ds(s * DPT, DPT)])
    plsc.subcore_barrier()

    for base, nk, nrows in HALVES:
        pltpu.sync_copy(src_hbm.at[pl.ds(wid * NCHS + base, nrows)],
                        sidx.at[pl.ds(0, nrows)])
        pltpu.sync_copy(dst_hbm.at[pl.ds(wid * NCHS + base, nrows)],
                        didx.at[pl.ds(0, nrows)])

        @pl.loop(0, nk // 2)
        def _(r):
            k = r * 2
            g0 = pltpu.async_copy(tab_hbm.at[sidx.at[k]], rows0, gsem0)
            g1 = pltpu.async_copy(tab_hbm.at[sidx.at[k + 1]], rows1, gsem1)
            g0.wait()
            s0 = pltpu.async_copy(rows0, acc.at[didx.at[k]], ssem0,
                                  add=True)
            g1.wait()
            s1 = pltpu.async_copy(rows1, acc.at[didx.at[k + 1]], ssem1,
                                  add=True)
            s0.wait()
            s1.wait()

    plsc.subcore_barrier()
    pltpu.sync_copy(acc.at[pl.ds(s * DPT, DPT)],
                    out_hbm.at[pl.ds(c * NACC + s * DPT, DPT)])


def _dinv_body(degp_ref, dinv_ref):
    deg = degp_ref[0] + degp_ref[1]
    dinv_ref[...] = jnp.where(
        deg > 0, lax.rsqrt(jnp.maximum(deg, 1e-12)), 0.0)


_dinv_call = pl.pallas_call(
    _dinv_body,
    out_shape=jax.ShapeDtypeStruct((NACC_D // 128, 128), jnp.float32),
)

_BR = 2000
_GRID = N // _BR


def _scale_body(x_ref, d_ref, o_ref):
    o_ref[...] = x_ref[...] * d_ref[...]


_scale_call = pl.pallas_call(
    _scale_body,
    grid=(_GRID,),
    in_specs=[
        pl.BlockSpec((_BR, F), lambda i: (i, 0)),
        pl.BlockSpec((_BR, 1), lambda i: (i, 0)),
    ],
    out_specs=pl.BlockSpec((_BR, F), lambda i: (i, 0)),
    out_shape=jax.ShapeDtypeStruct((N, F), jnp.float32),
)


def _mlp_body(p_ref, d_ref, w1_ref, b1_ref, w2_ref, y_ref):
    agg = (p_ref[0] + p_ref[1]) * d_ref[...]
    h = jnp.dot(agg, w1_ref[...], preferred_element_type=jnp.float32)
    h = jnp.maximum(h + b1_ref[...], 0.0)
    y = jnp.dot(h, w2_ref[...], preferred_element_type=jnp.float32)
    y_ref[...] = y * d_ref[...]


_mlp_call = pl.pallas_call(
    _mlp_body,
    grid=(_GRID,),
    in_specs=[
        pl.BlockSpec((2, _BR, F), lambda i: (0, i, 0)),
        pl.BlockSpec((_BR, 1), lambda i: (i, 0)),
        pl.BlockSpec((F, HID), lambda i: (0, 0)),
        pl.BlockSpec((1, HID), lambda i: (0, 0)),
        pl.BlockSpec((HID, F), lambda i: (0, 0)),
    ],
    out_specs=pl.BlockSpec((_BR, F), lambda i: (i, 0)),
    out_shape=jax.ShapeDtypeStruct((N, F), jnp.float32),
)


def _final_body(p_ref, d_ref, b2_ref, o_ref):
    o_ref[...] = (p_ref[0] + p_ref[1]) * d_ref[...] + b2_ref[...]


_final_call = pl.pallas_call(
    _final_body,
    grid=(_GRID,),
    in_specs=[
        pl.BlockSpec((2, _BR, F), lambda i: (0, i, 0)),
        pl.BlockSpec((_BR, 1), lambda i: (i, 0)),
        pl.BlockSpec((1, F), lambda i: (0, 0)),
    ],
    out_specs=pl.BlockSpec((_BR, F), lambda i: (i, 0)),
    out_shape=jax.ShapeDtypeStruct((N, F), jnp.float32),
)


def kernel(x, edge_index, W1, b1, W2, b2):
    loop = jnp.arange(N, dtype=jnp.int32)
    npad = E_PAD - (edge_index.shape[1] + N)
    pad_ids = jnp.arange(npad, dtype=jnp.int32)
    src = jnp.concatenate([edge_index[0], loop, pad_ids % N])
    dst = jnp.concatenate([edge_index[1], loop, DUMMY + pad_ids % (NACC - N)])
    nslabpad = NW * (NCHS - NCH) * CHB
    slab_ids = jnp.arange(nslabpad, dtype=jnp.int32)
    src2d = jnp.concatenate(
        [src.reshape(NW, NCH, CHB),
         (slab_ids % N).reshape(NW, NCHS - NCH, CHB)],
        axis=1).reshape(NW * NCHS, CHB)
    dst2d = jnp.concatenate(
        [dst.reshape(NW, NCH, CHB),
         (DUMMY + slab_ids % (NACC - N)).reshape(NW, NCHS - NCH, CHB)],
        axis=1).reshape(NW * NCHS, CHB)
    zd = jnp.zeros((DPT_D,), jnp.float32)
    zc = jnp.zeros((DPT, F), jnp.float32)

    degp = _deg_call(dst2d, zd)
    dinv2d = _dinv_call(degp.reshape(2, NACC_D // 128, 128))
    dinv_col = dinv2d.reshape(NACC_D, 1)[:N]
    xs = _scale_call(x, dinv_col)
    p1 = _agg_call(xs, src2d, dst2d, zc).reshape(2, NACC, F)
    y = _mlp_call(p1, dinv_col, W1, b1.reshape(1, HID), W2)
    p2 = _agg_call(y, src2d, dst2d, zc).reshape(2, NACC, F)
    return _final_call(p2, dinv_col, b2.reshape(1, F))
